# Initial kernel scaffold; baseline (speedup 1.0000x reference)
import jax
import jax.numpy as jnp
from jax.experimental import pallas as pl


def kernel(imgs, op_embs, num_transforms_embs, scale_embs, q, pnst):
    raise NotImplementedError("write your pallas kernel here")


# speedup vs baseline: 7.0126x; 7.0126x over previous
import functools

import jax
import jax.numpy as jnp
from jax.experimental import pallas as pl

_BB = 2048


def _body(op_ref, nte_ref, se_ref, q_ref, pnst_ref, ga_ref, r_ref, gc_ref,
          inds_ref, sc_ref, lp_ref, *, L, T, S, NH):
    f32 = jnp.float32

    ntl = jnp.dot(nte_ref[:], q_ref[:], preferred_element_type=f32)
    m0 = jnp.max(ntl, axis=0, keepdims=True)
    sh = ntl - m0
    lp_nt = sh - jnp.log(jnp.sum(jnp.exp(sh), axis=0, keepdims=True))

    ga = ga_ref[:]
    x = ga + ntl
    xm = jnp.max(x, axis=0, keepdims=True)
    io_nh = jax.lax.broadcasted_iota(jnp.int32, x.shape, 0)
    idx = jnp.min(jnp.where(x == xm, io_nh, NH), axis=0, keepdims=True)
    sel_nh = io_nh == idx
    lp_num = jnp.sum(jnp.where(sel_nh, lp_nt, 0.0), axis=0, keepdims=True)
    nt = jnp.sum(jnp.where(sel_nh, pnst_ref[:], 0), axis=0, keepdims=True)

    tblT = jax.lax.dot_general(se_ref[:], op_ref[:], (((1,), (1,)), ((), ())),
                               preferred_element_type=f32)
    tmax = jnp.max(tblT, axis=0, keepdims=True)
    lse = jnp.log(jnp.sum(jnp.exp(tblT - tmax), axis=0, keepdims=True))
    c_row = tmax + lse

    r = r_ref[:]
    io_s = jax.lax.broadcasted_iota(jnp.int32, (S, ga.shape[1]), 0)
    acc = lp_num
    for l in range(L):
        mask_l = nt <= l
        ind_l = jnp.where(mask_l, 0, r[l:l + 1, :])
        inds_ref[l:l + 1, :] = ind_l
        rows = jnp.where(ind_l == 0, tblT[:, 0:1], 0.0)
        logz = jnp.where(ind_l == 0, c_row[:, 0:1], 0.0)
        for k in range(1, T):
            sel = ind_l == k
            rows = rows + jnp.where(sel, tblT[:, k:k + 1], 0.0)
            logz = logz + jnp.where(sel, c_row[:, k:k + 1], 0.0)
        y = rows + gc_ref[l]
        ym = jnp.max(y, axis=0, keepdims=True)
        sc = jnp.min(jnp.where(y == ym, io_s, S), axis=0, keepdims=True)
        sc_ref[l:l + 1, :] = sc
        chosen = jnp.sum(jnp.where(io_s == sc, rows, 0.0), axis=0, keepdims=True)
        acc = acc + jnp.where(mask_l, 0.0, chosen - logz)
    lp_ref[:] = acc


def kernel(imgs, op_embs, num_transforms_embs, scale_embs, q, pnst):
    B = imgs.shape[0]
    T = op_embs.shape[0]
    S = scale_embs.shape[0]
    NH = num_transforms_embs.shape[0]
    L = NH - 1
    H = q.shape[0]

    skey = jax.random.key(42)
    kA, kB, kC = jax.random.split(skey, 3)
    gA = jax.random.gumbel(kA, (B, NH), jnp.float32)
    rinds = jax.random.randint(kB, (B, L), 0, T)
    gC = jax.random.gumbel(kC, (B * L, S), jnp.float32)

    gA_t = gA.T
    r_t = rinds.T
    gC_t = gC.reshape(B, L, S).transpose(1, 2, 0)
    q_c = q.reshape(H, 1)
    pnst_c = pnst.reshape(NH, 1)

    nblk = B // _BB
    full = lambda *shape: pl.BlockSpec(shape, lambda i: (0,) * len(shape))
    inds_t, sc_t, lp = pl.pallas_call(
        functools.partial(_body, L=L, T=T, S=S, NH=NH),
        grid=(nblk,),
        in_specs=[
            full(T, H),
            full(NH, H),
            full(S, H),
            full(H, 1),
            full(NH, 1),
            pl.BlockSpec((NH, _BB), lambda i: (0, i)),
            pl.BlockSpec((L, _BB), lambda i: (0, i)),
            pl.BlockSpec((L, S, _BB), lambda i: (0, 0, i)),
        ],
        out_specs=[
            pl.BlockSpec((L, _BB), lambda i: (0, i)),
            pl.BlockSpec((L, _BB), lambda i: (0, i)),
            pl.BlockSpec((1, _BB), lambda i: (0, i)),
        ],
        out_shape=[
            jax.ShapeDtypeStruct((L, B), jnp.int32),
            jax.ShapeDtypeStruct((L, B), jnp.int32),
            jax.ShapeDtypeStruct((1, B), jnp.float32),
        ],
    )(op_embs, num_transforms_embs, scale_embs, q_c, pnst_c, gA_t, r_t, gC_t)

    return (inds_t.T, sc_t.T, lp.reshape(B))
